# Initial kernel scaffold; baseline (speedup 1.0000x reference)
import jax
import jax.numpy as jnp
from jax.experimental import pallas as pl


def kernel(z_e_x, codebook):
    raise NotImplementedError("write your pallas kernel here")


# speedup vs baseline: 1.5067x; 1.5067x over previous
import functools

import jax
import jax.numpy as jnp
from jax import lax
from jax.experimental import pallas as pl
from jax.experimental.pallas import tpu as pltpu
from jax.experimental.pallas import tpu_sc as plsc

N = 32768
D = 64
K = 1024

BN = 512
NB = N // BN

NC = 2
NS = 16
NW = NC * NS
B_PER_W = N // NW
CHUNK = 128
NCHUNK = B_PER_W // CHUNK
DP = 128


def _argmin_body(z_ref, cbt_ref, out_ref):
    z = z_ref[...]
    zsq = jnp.sum(z ** 2, axis=1, keepdims=True)
    cbt = cbt_ref[...]
    cbsq = jnp.sum(cbt ** 2, axis=0, keepdims=True)
    mm = jnp.dot(z, cbt, preferred_element_type=jnp.float32)
    dists = zsq - 2.0 * mm + cbsq
    mn = jnp.min(dists, axis=1, keepdims=True)
    iota = lax.broadcasted_iota(jnp.int32, (BN, K), 1)
    idx = jnp.min(jnp.where(dists == mn, iota, K), axis=1, keepdims=True)
    out_ref[...] = idx.reshape(1, BN, 1)


_argmin_call = pl.pallas_call(
    _argmin_body,
    grid=(NB,),
    in_specs=[
        pl.BlockSpec((BN, D), lambda i: (i, 0)),
        pl.BlockSpec((D, K), lambda i: (0, 0)),
    ],
    out_specs=pl.BlockSpec((1, BN, 1), lambda i: (i, 0, 0)),
    out_shape=jax.ShapeDtypeStruct((NB, BN, 1), jnp.int32),
)


def _gather_body(table_hbm, idx_hbm, out_hbm, idx_v, rows_v, sem):
    wid = lax.axis_index("s") * NC + lax.axis_index("c")
    pltpu.sync_copy(idx_hbm.at[pl.ds(wid * NCHUNK, NCHUNK)], idx_v)
    cur = pltpu.async_copy(table_hbm.at[idx_v.at[0]], rows_v.at[0], sem)
    for j in range(NCHUNK):
        cur.wait()
        if j + 1 < NCHUNK:
            cur = pltpu.async_copy(
                table_hbm.at[idx_v.at[j + 1]], rows_v.at[(j + 1) % 2], sem)
        pltpu.sync_copy(
            rows_v.at[j % 2],
            out_hbm.at[pl.ds(wid * B_PER_W + j * CHUNK, CHUNK)],
        )


_gather_call = functools.partial(
    pl.kernel,
    out_type=jax.ShapeDtypeStruct((N, DP), jnp.float32),
    mesh=plsc.VectorSubcoreMesh(core_axis_name="c", subcore_axis_name="s"),
    scratch_types=[
        pltpu.VMEM((NCHUNK, CHUNK), jnp.int32),
        pltpu.VMEM((2, CHUNK, DP), jnp.float32),
        pltpu.SemaphoreType.DMA,
    ],
)(_gather_body)


def kernel(z_e_x, codebook):
    cbt = codebook.T
    cb_pad = jnp.pad(codebook, ((0, 0), (0, DP - D)))
    idx = _argmin_call(z_e_x, cbt).reshape(NW * NCHUNK, CHUNK)
    z_q_pad = _gather_call(cb_pad, idx)
    z_q_x = z_q_pad[:, :D]
    return (z_q_x, z_q_x)


# speedup vs baseline: 1.5494x; 1.0284x over previous
import functools

import jax
import jax.numpy as jnp
from jax import lax
from jax.experimental import pallas as pl
from jax.experimental.pallas import tpu as pltpu
from jax.experimental.pallas import tpu_sc as plsc

N = 32768
D = 64
K = 1024

NCHUNKS = 2
NT = N // NCHUNKS

BN = 512
NB = NT // BN

NC = 2
NS = 16
NW = NC * NS
B_PER_W = NT // NW
CHUNK = 128
NCHUNK = B_PER_W // CHUNK
ROWS_PER_W = B_PER_W // BN
DP = 128


def _argmin_body(zt_ref, cbt_ref, out_ref):
    zt = zt_ref[...]
    zsq = jnp.sum(zt * zt, axis=0, keepdims=True)
    cbt = cbt_ref[...]
    ones = jnp.ones((D, 1), jnp.float32)
    cbsq = lax.dot_general(
        cbt * cbt, ones, (((0,), (0,)), ((), ())),
        preferred_element_type=jnp.float32)
    mmneg2 = lax.dot_general(
        cbt * -2.0, zt, (((0,), (0,)), ((), ())),
        preferred_element_type=jnp.float32)
    dists = (zsq + mmneg2) + cbsq
    mn = jnp.min(dists, axis=0, keepdims=True)
    iota = lax.broadcasted_iota(jnp.int32, (K, BN), 0)
    idx = jnp.min(jnp.where(dists == mn, iota, K), axis=0, keepdims=True)
    out_ref[pl.ds(pl.program_id(0), 1), :] = idx


_argmin_call = pl.pallas_call(
    _argmin_body,
    grid=(NB,),
    in_specs=[
        pl.BlockSpec((D, BN), lambda i: (0, i)),
        pl.BlockSpec((D, K), lambda i: (0, 0)),
    ],
    out_specs=pl.BlockSpec((NB, BN), lambda i: (0, 0)),
    out_shape=jax.ShapeDtypeStruct((NB, BN), jnp.int32),
)


def _gather_body(table_hbm, idx_hbm, out_hbm, idx_v, rows_v, sem):
    wid = lax.axis_index("s") * NC + lax.axis_index("c")
    pltpu.sync_copy(idx_hbm.at[pl.ds(wid * ROWS_PER_W, ROWS_PER_W)], idx_v)
    chunks_per_row = BN // CHUNK

    def idx_chunk(j):
        return idx_v.at[j // chunks_per_row,
                        pl.ds((j % chunks_per_row) * CHUNK, CHUNK)]

    cur = pltpu.async_copy(table_hbm.at[idx_chunk(0)], rows_v.at[0], sem)
    for j in range(NCHUNK):
        cur.wait()
        if j + 1 < NCHUNK:
            cur = pltpu.async_copy(
                table_hbm.at[idx_chunk(j + 1)], rows_v.at[(j + 1) % 2], sem)
        pltpu.sync_copy(
            rows_v.at[j % 2],
            out_hbm.at[pl.ds(wid * B_PER_W + j * CHUNK, CHUNK)],
        )


_gather_call = functools.partial(
    pl.kernel,
    out_type=jax.ShapeDtypeStruct((NT, DP), jnp.float32),
    mesh=plsc.VectorSubcoreMesh(core_axis_name="c", subcore_axis_name="s"),
    scratch_types=[
        pltpu.VMEM((ROWS_PER_W, BN), jnp.int32),
        pltpu.VMEM((2, CHUNK, DP), jnp.float32),
        pltpu.SemaphoreType.DMA,
    ],
)(_gather_body)


def kernel(z_e_x, codebook):
    zt = z_e_x.T
    cbt = codebook.T
    cb_pad = jnp.pad(codebook, ((0, 0), (0, DP - D)))
    parts = []
    for c in range(NCHUNKS):
        idx_c = _argmin_call(lax.slice(zt, (0, c * NT), (D, (c + 1) * NT)),
                             cbt)
        parts.append(_gather_call(cb_pad, idx_c))
    z_q_x = jnp.concatenate(parts, axis=0)[:, :D]
    return (z_q_x, z_q_x)


# speedup vs baseline: 1.6399x; 1.0584x over previous
import functools

import jax
import jax.numpy as jnp
from jax import lax
from jax.experimental import pallas as pl
from jax.experimental.pallas import tpu as pltpu
from jax.experimental.pallas import tpu_sc as plsc

N = 32768
D = 64
K = 1024

NCHUNKS = 2
NT = N // NCHUNKS

BN = 512
NB = NT // BN

NC = 2
NS = 16
NW = NC * NS
B_PER_W = NT // NW
CHUNK = 128
NCHUNK = B_PER_W // CHUNK
ROWS_PER_W = B_PER_W // BN
DP = 128


def _argmin_body(zt_ref, cbt_ref, out_ref):
    zt = zt_ref[...]
    zsq = jnp.sum(zt * zt, axis=0, keepdims=True)
    cbt = cbt_ref[...]
    ones = jnp.ones((D, 1), jnp.float32)
    cbsq = lax.dot_general(
        cbt * cbt, ones, (((0,), (0,)), ((), ())),
        preferred_element_type=jnp.float32)
    mmneg2 = lax.dot_general(
        cbt * -2.0, zt, (((0,), (0,)), ((), ())),
        preferred_element_type=jnp.float32)
    dists = (zsq + mmneg2) + cbsq
    mn = jnp.min(dists, axis=0, keepdims=True)
    iota = lax.broadcasted_iota(jnp.int32, (K, BN), 0)
    idx = jnp.min(jnp.where(dists == mn, iota, K), axis=0, keepdims=True)
    out_ref[pl.ds(pl.program_id(0), 1), :] = idx


def _make_argmin_call(c):
    return pl.pallas_call(
        _argmin_body,
        grid=(NB,),
        in_specs=[
            pl.BlockSpec((D, BN), lambda i, c=c: (0, c * NB + i)),
            pl.BlockSpec((D, K), lambda i: (0, 0)),
        ],
        out_specs=pl.BlockSpec((NB, BN), lambda i: (0, 0)),
        out_shape=jax.ShapeDtypeStruct((NB, BN), jnp.int32),
    )


_argmin_calls = [_make_argmin_call(c) for c in range(NCHUNKS)]


def _gather_body(table_hbm, idx_hbm, out_hbm, idx_v, rows_v, sem):
    wid = lax.axis_index("s") * NC + lax.axis_index("c")
    pltpu.sync_copy(idx_hbm.at[pl.ds(wid * ROWS_PER_W, ROWS_PER_W)], idx_v)
    chunks_per_row = BN // CHUNK

    def idx_chunk(j):
        return idx_v.at[j // chunks_per_row,
                        pl.ds((j % chunks_per_row) * CHUNK, CHUNK)]

    cur = pltpu.async_copy(table_hbm.at[idx_chunk(0)], rows_v.at[0], sem)
    for j in range(NCHUNK):
        cur.wait()
        if j + 1 < NCHUNK:
            cur = pltpu.async_copy(
                table_hbm.at[idx_chunk(j + 1)], rows_v.at[(j + 1) % 2], sem)
        pltpu.sync_copy(
            rows_v.at[j % 2],
            out_hbm.at[pl.ds(wid * B_PER_W + j * CHUNK, CHUNK)],
        )


_gather_call = functools.partial(
    pl.kernel,
    out_type=jax.ShapeDtypeStruct((NT, DP), jnp.float32),
    mesh=plsc.VectorSubcoreMesh(core_axis_name="c", subcore_axis_name="s"),
    scratch_types=[
        pltpu.VMEM((ROWS_PER_W, BN), jnp.int32),
        pltpu.VMEM((2, CHUNK, DP), jnp.float32),
        pltpu.SemaphoreType.DMA,
    ],
)(_gather_body)


def kernel(z_e_x, codebook):
    zt = z_e_x.T
    cbt = codebook.T
    cb_pad = jnp.pad(codebook, ((0, 0), (0, DP - D)))
    parts = []
    for c in range(NCHUNKS):
        idx_c = _argmin_calls[c](zt, cbt)
        parts.append(_gather_call(cb_pad, idx_c)[:, :D])
    z_q_x = jnp.concatenate(parts, axis=0)
    return (z_q_x, z_q_x)


# speedup vs baseline: 2.2765x; 1.3881x over previous
import jax
import jax.numpy as jnp
from jax import lax
from jax.experimental import pallas as pl

N = 32768
D = 64
K = 1024

BN = 512
NB = N // BN


def _vq_body(zt_ref, cbt_ref, out_ref):
    zt = zt_ref[...]
    zsq = jnp.sum(zt * zt, axis=0, keepdims=True)
    cbt = cbt_ref[...]
    ones = jnp.ones((D, 1), jnp.float32)
    cbsq = lax.dot_general(
        cbt * cbt, ones, (((0,), (0,)), ((), ())),
        preferred_element_type=jnp.float32)
    mmneg2 = lax.dot_general(
        cbt * -2.0, zt, (((0,), (0,)), ((), ())),
        preferred_element_type=jnp.float32)
    dists = (zsq + mmneg2) + cbsq
    mn = jnp.min(dists, axis=0, keepdims=True)
    iota = lax.broadcasted_iota(jnp.int32, (K, BN), 0)
    t = jnp.where(dists == mn, iota, K)
    idx = jnp.min(t, axis=0, keepdims=True)
    onehot = jnp.where(t == idx, 1.0, 0.0)
    out_ref[...] = lax.dot_general(
        cbt, onehot, (((1,), (0,)), ((), ())),
        preferred_element_type=jnp.float32)


_vq_call = pl.pallas_call(
    _vq_body,
    grid=(NB,),
    in_specs=[
        pl.BlockSpec((D, BN), lambda i: (0, i)),
        pl.BlockSpec((D, K), lambda i: (0, 0)),
    ],
    out_specs=pl.BlockSpec((D, BN), lambda i: (0, i)),
    out_shape=jax.ShapeDtypeStruct((D, N), jnp.float32),
)


def kernel(z_e_x, codebook):
    z_q_t = _vq_call(z_e_x.T, codebook.T)
    z_q_x = z_q_t.T
    return (z_q_x, z_q_x)
